# Initial kernel scaffold; baseline (speedup 1.0000x reference)
import jax
import jax.numpy as jnp
from jax.experimental import pallas as pl


def kernel(rl_logprobs, kge_logprobs, pred_indices, alphas):
    raise NotImplementedError("write your pallas kernel here")


# speedup vs baseline: 109.4332x; 109.4332x over previous
import jax
import jax.numpy as jnp
from jax import lax
from jax.experimental import pallas as pl
from jax.experimental.pallas import tpu as pltpu
from jax.experimental.pallas import tpu_sc as plsc

N_PRED = 100000
TBL_PAD = 100352
NC = 2
NS = 16
NW = NC * NS
LANES = 16
CHUNK = 2048


def _sigmoid_body(x_ref, o_ref):
    o_ref[...] = jax.nn.sigmoid(x_ref[...])


def _bridge_body(tbl_hbm, idx_hbm, rl_hbm, kge_hbm, out_hbm,
                 tbl_v, idx_v, rl_v, kge_v, out_v, n_chunks):
    wid = lax.axis_index("s") * NC + lax.axis_index("c")
    per_w = n_chunks * CHUNK
    base = wid * per_w
    pltpu.sync_copy(tbl_hbm, tbl_v)

    def chunk_body(g, carry):
        off = base + g * CHUNK
        pltpu.sync_copy(idx_hbm.at[pl.ds(off, CHUNK)], idx_v)
        pltpu.sync_copy(rl_hbm.at[pl.ds(off, CHUNK)], rl_v)
        pltpu.sync_copy(kge_hbm.at[pl.ds(off, CHUNK)], kge_v)

        @plsc.parallel_loop(0, CHUNK, LANES, unroll=4)
        def _inner(j):
            iv = idx_v[pl.ds(j, LANES)]
            a = plsc.load_gather(tbl_v, [iv])
            r = rl_v[pl.ds(j, LANES)]
            k = kge_v[pl.ds(j, LANES)]
            out_v[pl.ds(j, LANES)] = k + a * (r - k)

        pltpu.sync_copy(out_v, out_hbm.at[pl.ds(off, CHUNK)])
        return carry

    lax.fori_loop(0, n_chunks, chunk_body, 0)


def kernel(rl_logprobs, kge_logprobs, pred_indices, alphas):
    B, K = rl_logprobs.shape
    BK = B * K
    n_chunks = BK // (NW * CHUNK)
    assert n_chunks * NW * CHUNK == BK

    alphas_p = jnp.pad(alphas, (0, TBL_PAD - N_PRED))
    sig_tbl = pl.pallas_call(
        _sigmoid_body,
        out_shape=jax.ShapeDtypeStruct((TBL_PAD // 128, 128), jnp.float32),
    )(alphas_p.reshape(TBL_PAD // 128, 128)).reshape(TBL_PAD)

    idx_flat = pred_indices.reshape(BK).astype(jnp.int32)
    rl_flat = rl_logprobs.reshape(BK)
    kge_flat = kge_logprobs.reshape(BK)

    import functools
    body = functools.partial(_bridge_body, n_chunks=n_chunks)
    out_flat = pl.kernel(
        body,
        out_type=jax.ShapeDtypeStruct((BK,), jnp.float32),
        mesh=plsc.VectorSubcoreMesh(
            core_axis_name="c", subcore_axis_name="s",
            num_cores=NC, num_subcores=NS),
        compiler_params=pltpu.CompilerParams(needs_layout_passes=False),
        scratch_types=[
            pltpu.VMEM((TBL_PAD,), jnp.float32),
            pltpu.VMEM((CHUNK,), jnp.int32),
            pltpu.VMEM((CHUNK,), jnp.float32),
            pltpu.VMEM((CHUNK,), jnp.float32),
            pltpu.VMEM((CHUNK,), jnp.float32),
        ],
    )(sig_tbl, idx_flat, rl_flat, kge_flat)
    return out_flat.reshape(B, K)


# speedup vs baseline: 159.1016x; 1.4539x over previous
import functools

import jax
import jax.numpy as jnp
from jax import lax
from jax.experimental import pallas as pl
from jax.experimental.pallas import tpu as pltpu
from jax.experimental.pallas import tpu_sc as plsc

N_PRED = 100000
TBL_PAD = 100352
NC = 2
NS = 16
NW = NC * NS
LANES = 16
CHUNK = 3200


def _sigmoid_body(x_ref, o_ref):
    o_ref[...] = jax.nn.sigmoid(x_ref[...])


def _bridge_body(tbl_hbm, idx_hbm, rl_hbm, kge_hbm, out_hbm,
                 tbl_v, idx_v, rl_v, kge_v, out_v,
                 in_sem0, in_sem1, out_sem0, out_sem1, n_chunks):
    wid = lax.axis_index("s") * NC + lax.axis_index("c")
    per_w = n_chunks * CHUNK
    base = wid * per_w
    in_sems = (in_sem0, in_sem1)
    out_sems = (out_sem0, out_sem1)

    pltpu.sync_copy(tbl_hbm, tbl_v)

    def start_in(g, slot):
        off = base + g * CHUNK
        pltpu.async_copy(idx_hbm.at[pl.ds(off, CHUNK)], idx_v.at[slot],
                         in_sems[slot])
        pltpu.async_copy(rl_hbm.at[pl.ds(off, CHUNK)], rl_v.at[slot],
                         in_sems[slot])
        pltpu.async_copy(kge_hbm.at[pl.ds(off, CHUNK)], kge_v.at[slot],
                         in_sems[slot])

    def wait_in(slot):
        pltpu.make_async_copy(idx_hbm.at[pl.ds(0, CHUNK)], idx_v.at[slot],
                              in_sems[slot]).wait()
        pltpu.make_async_copy(rl_hbm.at[pl.ds(0, CHUNK)], rl_v.at[slot],
                              in_sems[slot]).wait()
        pltpu.make_async_copy(kge_hbm.at[pl.ds(0, CHUNK)], kge_v.at[slot],
                              in_sems[slot]).wait()

    def start_out(g, slot):
        off = base + g * CHUNK
        pltpu.async_copy(out_v.at[slot], out_hbm.at[pl.ds(off, CHUNK)],
                         out_sems[slot])

    def wait_out(slot):
        pltpu.make_async_copy(out_v.at[slot], out_hbm.at[pl.ds(0, CHUNK)],
                              out_sems[slot]).wait()

    def compute(slot):
        @plsc.parallel_loop(0, CHUNK, LANES, unroll=4)
        def _inner(j):
            iv = idx_v[slot, pl.ds(j, LANES)]
            a = plsc.load_gather(tbl_v, [iv])
            r = rl_v[slot, pl.ds(j, LANES)]
            k = kge_v[slot, pl.ds(j, LANES)]
            out_v[slot, pl.ds(j, LANES)] = k + a * (r - k)

    n_pairs = n_chunks // 2
    start_in(0, 0)

    def pair_body(p, carry):
        g0 = p * 2
        start_in(g0 + 1, 1)
        wait_in(0)

        @pl.when(p > 0)
        def _():
            wait_out(0)

        compute(0)
        start_out(g0, 0)

        @pl.when(p + 1 < n_pairs)
        def _():
            start_in(g0 + 2, 0)

        wait_in(1)

        @pl.when(p > 0)
        def _():
            wait_out(1)

        compute(1)
        start_out(g0 + 1, 1)
        return carry

    lax.fori_loop(0, n_pairs, pair_body, 0)
    wait_out(0)
    wait_out(1)


def kernel(rl_logprobs, kge_logprobs, pred_indices, alphas):
    B, K = rl_logprobs.shape
    BK = B * K
    n_chunks = BK // (NW * CHUNK)
    assert n_chunks * NW * CHUNK == BK and n_chunks % 2 == 0

    alphas_p = jnp.pad(alphas, (0, TBL_PAD - N_PRED))
    sig_tbl = pl.pallas_call(
        _sigmoid_body,
        out_shape=jax.ShapeDtypeStruct((TBL_PAD // 128, 128), jnp.float32),
    )(alphas_p.reshape(TBL_PAD // 128, 128)).reshape(TBL_PAD)

    idx_flat = pred_indices.reshape(BK).astype(jnp.int32)
    rl_flat = rl_logprobs.reshape(BK)
    kge_flat = kge_logprobs.reshape(BK)

    body = functools.partial(_bridge_body, n_chunks=n_chunks)
    out_flat = pl.kernel(
        body,
        out_type=jax.ShapeDtypeStruct((BK,), jnp.float32),
        mesh=plsc.VectorSubcoreMesh(
            core_axis_name="c", subcore_axis_name="s",
            num_cores=NC, num_subcores=NS),
        compiler_params=pltpu.CompilerParams(needs_layout_passes=False),
        scratch_types=[
            pltpu.VMEM((TBL_PAD,), jnp.float32),
            pltpu.VMEM((2, CHUNK), jnp.int32),
            pltpu.VMEM((2, CHUNK), jnp.float32),
            pltpu.VMEM((2, CHUNK), jnp.float32),
            pltpu.VMEM((2, CHUNK), jnp.float32),
            pltpu.SemaphoreType.DMA,
            pltpu.SemaphoreType.DMA,
            pltpu.SemaphoreType.DMA,
            pltpu.SemaphoreType.DMA,
        ],
    )(sig_tbl, idx_flat, rl_flat, kge_flat)
    return out_flat.reshape(B, K)


# speedup vs baseline: 234.0914x; 1.4713x over previous
import functools

import jax
import jax.numpy as jnp
from jax import lax
from jax.experimental import pallas as pl
from jax.experimental.pallas import tpu as pltpu
from jax.experimental.pallas import tpu_sc as plsc

N_PRED = 100000
TBL_PAD = 100352
NC = 2
NS = 16
NW = NC * NS
LANES = 16
ROWS = 8


def _sigmoid_body(x_ref, o_ref):
    o_ref[...] = jax.nn.sigmoid(x_ref[...])


def _bridge_body(tbl_hbm, idx_hbm, rl_hbm, kge_hbm, out_hbm,
                 tbl_v, idx_v, rl_v, kge_v, out_v,
                 in_sem0, in_sem1, out_sem0, out_sem1, n_chunks, K):
    wid = lax.axis_index("s") * NC + lax.axis_index("c")
    base = wid * n_chunks * ROWS
    in_sems = (in_sem0, in_sem1)
    out_sems = (out_sem0, out_sem1)
    col_starts = list(range(0, K - LANES + 1, LANES))
    if col_starts[-1] + LANES < K:
        col_starts.append(K - LANES)

    pltpu.sync_copy(tbl_hbm, tbl_v)

    def start_in(g, slot):
        r0 = base + g * ROWS
        pltpu.async_copy(idx_hbm.at[pl.ds(r0, ROWS), :], idx_v.at[slot],
                         in_sems[slot])
        pltpu.async_copy(rl_hbm.at[pl.ds(r0, ROWS), :], rl_v.at[slot],
                         in_sems[slot])
        pltpu.async_copy(kge_hbm.at[pl.ds(r0, ROWS), :], kge_v.at[slot],
                         in_sems[slot])

    def wait_in(slot):
        pltpu.make_async_copy(idx_hbm.at[pl.ds(0, ROWS), :], idx_v.at[slot],
                              in_sems[slot]).wait()
        pltpu.make_async_copy(rl_hbm.at[pl.ds(0, ROWS), :], rl_v.at[slot],
                              in_sems[slot]).wait()
        pltpu.make_async_copy(kge_hbm.at[pl.ds(0, ROWS), :], kge_v.at[slot],
                              in_sems[slot]).wait()

    def start_out(g, slot):
        r0 = base + g * ROWS
        pltpu.async_copy(out_v.at[slot], out_hbm.at[pl.ds(r0, ROWS), :],
                         out_sems[slot])

    def wait_out(slot):
        pltpu.make_async_copy(out_v.at[slot], out_hbm.at[pl.ds(0, ROWS), :],
                              out_sems[slot]).wait()

    def compute(slot):
        @plsc.parallel_loop(0, ROWS, 1)
        def _row(r):
            for c in col_starts:
                iv = idx_v[slot, r, pl.ds(c, LANES)]
                a = plsc.load_gather(tbl_v, [iv >> 7, iv & 127])
                rr = rl_v[slot, r, pl.ds(c, LANES)]
                kk = kge_v[slot, r, pl.ds(c, LANES)]
                out_v[slot, r, pl.ds(c, LANES)] = kk + a * (rr - kk)

    n_pairs = n_chunks // 2
    start_in(0, 0)

    def pair_body(p, carry):
        g0 = p * 2
        start_in(g0 + 1, 1)
        wait_in(0)

        @pl.when(p > 0)
        def _():
            wait_out(0)

        compute(0)
        start_out(g0, 0)

        @pl.when(p + 1 < n_pairs)
        def _():
            start_in(g0 + 2, 0)

        wait_in(1)

        @pl.when(p > 0)
        def _():
            wait_out(1)

        compute(1)
        start_out(g0 + 1, 1)
        return carry

    lax.fori_loop(0, n_pairs, pair_body, 0)
    wait_out(0)
    wait_out(1)


def kernel(rl_logprobs, kge_logprobs, pred_indices, alphas):
    B, K = rl_logprobs.shape
    n_chunks = B // (NW * ROWS)
    assert n_chunks * NW * ROWS == B and n_chunks % 2 == 0

    alphas_p = jnp.pad(alphas, (0, TBL_PAD - N_PRED))
    sig_tbl = pl.pallas_call(
        _sigmoid_body,
        out_shape=jax.ShapeDtypeStruct((TBL_PAD // 128, 128), jnp.float32),
    )(alphas_p.reshape(TBL_PAD // 128, 128))

    idx2d = pred_indices.astype(jnp.int32)

    body = functools.partial(_bridge_body, n_chunks=n_chunks, K=K)
    out = pl.kernel(
        body,
        out_type=jax.ShapeDtypeStruct((B, K), jnp.float32),
        mesh=plsc.VectorSubcoreMesh(
            core_axis_name="c", subcore_axis_name="s",
            num_cores=NC, num_subcores=NS),
        compiler_params=pltpu.CompilerParams(
            needs_layout_passes=False, use_tc_tiling_on_sc=True),
        scratch_types=[
            pltpu.VMEM((TBL_PAD // 128, 128), jnp.float32),
            pltpu.VMEM((2, ROWS, K), jnp.int32),
            pltpu.VMEM((2, ROWS, K), jnp.float32),
            pltpu.VMEM((2, ROWS, K), jnp.float32),
            pltpu.VMEM((2, ROWS, K), jnp.float32),
            pltpu.SemaphoreType.DMA,
            pltpu.SemaphoreType.DMA,
            pltpu.SemaphoreType.DMA,
            pltpu.SemaphoreType.DMA,
        ],
    )(sig_tbl, idx2d, rl_logprobs, kge_logprobs)
    return out


# speedup vs baseline: 458.9947x; 1.9607x over previous
import functools

import jax
import jax.numpy as jnp
from jax import lax
from jax.experimental import pallas as pl
from jax.experimental.pallas import tpu as pltpu
from jax.experimental.pallas import tpu_sc as plsc

N_PRED = 100000
TBL_PAD = 100352
NC = 2
NS = 16
NW = NC * NS
LANES = 16
RH = 8
CW = 256


def _sigmoid_body(x_ref, o_ref):
    o_ref[...] = jax.nn.sigmoid(x_ref[...])


def _bridge_body(tbl_hbm, idx_hbm, rl_hbm, kge_hbm, out_hbm,
                 tbl_v, idx_v, rl_v, kge_v, out_v,
                 in_sem0, in_sem1, out_sem0, out_sem1, n_per, ncg_shift):
    wid = lax.axis_index("s") * NC + lax.axis_index("c")
    base = wid * n_per
    in_sems = (in_sem0, in_sem1)
    out_sems = (out_sem0, out_sem1)
    ncg_mask = (1 << ncg_shift) - 1

    pltpu.sync_copy(tbl_hbm, tbl_v)

    def slab(g):
        q = base + g
        return (q >> ncg_shift) * RH, (q & ncg_mask) * CW

    def start_in(g, slot):
        r0, c0 = slab(g)
        pltpu.async_copy(idx_hbm.at[pl.ds(r0, RH), pl.ds(c0, CW)],
                         idx_v.at[slot], in_sems[slot])
        pltpu.async_copy(rl_hbm.at[pl.ds(r0, RH), pl.ds(c0, CW)],
                         rl_v.at[slot], in_sems[slot])
        pltpu.async_copy(kge_hbm.at[pl.ds(r0, RH), pl.ds(c0, CW)],
                         kge_v.at[slot], in_sems[slot])

    def wait_in(slot):
        pltpu.make_async_copy(idx_hbm.at[pl.ds(0, RH), pl.ds(0, CW)],
                              idx_v.at[slot], in_sems[slot]).wait()
        pltpu.make_async_copy(rl_hbm.at[pl.ds(0, RH), pl.ds(0, CW)],
                              rl_v.at[slot], in_sems[slot]).wait()
        pltpu.make_async_copy(kge_hbm.at[pl.ds(0, RH), pl.ds(0, CW)],
                              kge_v.at[slot], in_sems[slot]).wait()

    def start_out(g, slot):
        r0, c0 = slab(g)
        pltpu.async_copy(out_v.at[slot],
                         out_hbm.at[pl.ds(r0, RH), pl.ds(c0, CW)],
                         out_sems[slot])

    def wait_out(slot):
        pltpu.make_async_copy(out_v.at[slot],
                              out_hbm.at[pl.ds(0, RH), pl.ds(0, CW)],
                              out_sems[slot]).wait()

    def compute(slot):
        @plsc.parallel_loop(0, RH, 1)
        def _row(r):
            for c in range(0, CW, LANES):
                iv = idx_v[slot, r, pl.ds(c, LANES)]
                a = plsc.load_gather(tbl_v, [iv >> 7, iv & 127])
                rr = rl_v[slot, r, pl.ds(c, LANES)]
                kk = kge_v[slot, r, pl.ds(c, LANES)]
                out_v[slot, r, pl.ds(c, LANES)] = kk + a * (rr - kk)

    n_pairs = n_per // 2
    start_in(0, 0)

    def pair_body(p, carry):
        g0 = p * 2
        start_in(g0 + 1, 1)
        wait_in(0)

        @pl.when(p > 0)
        def _():
            wait_out(0)

        compute(0)
        start_out(g0, 0)

        @pl.when(p + 1 < n_pairs)
        def _():
            start_in(g0 + 2, 0)

        wait_in(1)

        @pl.when(p > 0)
        def _():
            wait_out(1)

        compute(1)
        start_out(g0 + 1, 1)
        return carry

    lax.fori_loop(0, n_pairs, pair_body, 0)
    wait_out(0)
    wait_out(1)


def kernel(rl_logprobs, kge_logprobs, pred_indices, alphas):
    B, K = rl_logprobs.shape
    assert K % RH == 0 and B % CW == 0
    ncg = B // CW
    ncg_shift = ncg.bit_length() - 1
    assert (1 << ncg_shift) == ncg
    n_chunks = (K // RH) * ncg
    n_per = n_chunks // NW
    assert n_per * NW == n_chunks and n_per % 2 == 0

    alphas_p = jnp.pad(alphas, (0, TBL_PAD - N_PRED))
    sig_tbl = pl.pallas_call(
        _sigmoid_body,
        out_shape=jax.ShapeDtypeStruct((TBL_PAD // 128, 128), jnp.float32),
    )(alphas_p.reshape(TBL_PAD // 128, 128))

    idx_t = pred_indices.astype(jnp.int32).T
    rl_t = rl_logprobs.T
    kge_t = kge_logprobs.T

    body = functools.partial(_bridge_body, n_per=n_per, ncg_shift=ncg_shift)
    out_t = pl.kernel(
        body,
        out_type=jax.ShapeDtypeStruct((K, B), jnp.float32),
        mesh=plsc.VectorSubcoreMesh(
            core_axis_name="c", subcore_axis_name="s",
            num_cores=NC, num_subcores=NS),
        compiler_params=pltpu.CompilerParams(
            needs_layout_passes=False, use_tc_tiling_on_sc=True),
        scratch_types=[
            pltpu.VMEM((TBL_PAD // 128, 128), jnp.float32),
            pltpu.VMEM((2, RH, CW), jnp.int32),
            pltpu.VMEM((2, RH, CW), jnp.float32),
            pltpu.VMEM((2, RH, CW), jnp.float32),
            pltpu.VMEM((2, RH, CW), jnp.float32),
            pltpu.SemaphoreType.DMA,
            pltpu.SemaphoreType.DMA,
            pltpu.SemaphoreType.DMA,
            pltpu.SemaphoreType.DMA,
        ],
    )(sig_tbl, idx_t, rl_t, kge_t)
    return out_t.T


# speedup vs baseline: 550.4748x; 1.1993x over previous
import functools

import jax
import jax.numpy as jnp
from jax import lax
from jax.experimental import pallas as pl
from jax.experimental.pallas import tpu as pltpu
from jax.experimental.pallas import tpu_sc as plsc

N_PRED = 100000
TBL_PAD = 100352
NC = 2
NS = 16
NW = NC * NS
LANES = 16
RH = 8
CW = 256
N_IN = 4
N_OUT = 2


def _sigmoid_body(x_ref, o_ref):
    o_ref[...] = jax.nn.sigmoid(x_ref[...])


def _bridge_body(tbl_hbm, idx_hbm, rl_hbm, kge_hbm, out_hbm,
                 tbl_v, idx_v, rl_v, kge_v, out_v,
                 in_sems, out_sems, n_per, ncg_shift):
    wid = lax.axis_index("s") * NC + lax.axis_index("c")
    base = wid * n_per
    ncg_mask = (1 << ncg_shift) - 1

    pltpu.sync_copy(tbl_hbm, tbl_v)

    def slab(g):
        q = base + g
        return (q >> ncg_shift) * RH, (q & ncg_mask) * CW

    def start_in(g, slot):
        r0, c0 = slab(g)
        pltpu.async_copy(idx_hbm.at[pl.ds(r0, RH), pl.ds(c0, CW)],
                         idx_v.at[slot], in_sems[slot])
        pltpu.async_copy(rl_hbm.at[pl.ds(r0, RH), pl.ds(c0, CW)],
                         rl_v.at[slot], in_sems[slot])
        pltpu.async_copy(kge_hbm.at[pl.ds(r0, RH), pl.ds(c0, CW)],
                         kge_v.at[slot], in_sems[slot])

    def wait_in(slot):
        pltpu.make_async_copy(idx_hbm.at[pl.ds(0, RH), pl.ds(0, CW)],
                              idx_v.at[slot], in_sems[slot]).wait()
        pltpu.make_async_copy(rl_hbm.at[pl.ds(0, RH), pl.ds(0, CW)],
                              rl_v.at[slot], in_sems[slot]).wait()
        pltpu.make_async_copy(kge_hbm.at[pl.ds(0, RH), pl.ds(0, CW)],
                              kge_v.at[slot], in_sems[slot]).wait()

    def start_out(g, slot):
        r0, c0 = slab(g)
        pltpu.async_copy(out_v.at[slot],
                         out_hbm.at[pl.ds(r0, RH), pl.ds(c0, CW)],
                         out_sems[slot])

    def wait_out(slot):
        pltpu.make_async_copy(out_v.at[slot],
                              out_hbm.at[pl.ds(0, RH), pl.ds(0, CW)],
                              out_sems[slot]).wait()

    def compute(in_slot, out_slot):
        @plsc.parallel_loop(0, RH, 1)
        def _row(r):
            for c in range(0, CW, LANES):
                iv = idx_v[in_slot, r, pl.ds(c, LANES)]
                a = plsc.load_gather(tbl_v, [iv >> 7, iv & 127])
                rr = rl_v[in_slot, r, pl.ds(c, LANES)]
                kk = kge_v[in_slot, r, pl.ds(c, LANES)]
                out_v[out_slot, r, pl.ds(c, LANES)] = kk + a * (rr - kk)

    n_main = (n_per // N_IN) * N_IN
    n_tail = n_per - n_main
    assert n_tail < N_IN and n_tail % N_OUT == 0

    for s in range(N_IN - 1):
        start_in(s, s)

    def group_body(p, carry):
        for b in range(N_IN):
            g = p * N_IN + b
            pre_g = g + N_IN - 1

            @pl.when(pre_g < n_per)
            def _():
                start_in(pre_g, (b + N_IN - 1) % N_IN)

            wait_in(b)

            @pl.when(g >= N_OUT)
            def _():
                wait_out(b % N_OUT)

            compute(b, b % N_OUT)
            start_out(g, b % N_OUT)
        return carry

    lax.fori_loop(0, n_main // N_IN, group_body, 0)

    for t in range(n_tail):
        g = n_main + t
        wait_in(g % N_IN)
        wait_out(g % N_OUT)
        compute(g % N_IN, g % N_OUT)
        start_out(g, g % N_OUT)

    for s in range(N_OUT):
        wait_out(s)


def kernel(rl_logprobs, kge_logprobs, pred_indices, alphas):
    B, K = rl_logprobs.shape
    assert K % RH == 0 and B % CW == 0
    ncg = B // CW
    ncg_shift = ncg.bit_length() - 1
    assert (1 << ncg_shift) == ncg
    n_chunks = (K // RH) * ncg
    n_per = n_chunks // NW
    assert n_per * NW == n_chunks

    alphas_p = jnp.pad(alphas, (0, TBL_PAD - N_PRED))
    sig_tbl = pl.pallas_call(
        _sigmoid_body,
        out_shape=jax.ShapeDtypeStruct((TBL_PAD // 128, 128), jnp.float32),
    )(alphas_p.reshape(TBL_PAD // 128, 128))

    idx_t = pred_indices.astype(jnp.int32).T
    rl_t = rl_logprobs.T
    kge_t = kge_logprobs.T

    def body(tbl_hbm, idx_hbm, rl_hbm, kge_hbm, out_hbm,
             tbl_v, idx_v, rl_v, kge_v, out_v, *sems):
        _bridge_body(tbl_hbm, idx_hbm, rl_hbm, kge_hbm, out_hbm,
                     tbl_v, idx_v, rl_v, kge_v, out_v,
                     sems[:N_IN], sems[N_IN:], n_per, ncg_shift)

    out_t = pl.kernel(
        body,
        out_type=jax.ShapeDtypeStruct((K, B), jnp.float32),
        mesh=plsc.VectorSubcoreMesh(
            core_axis_name="c", subcore_axis_name="s",
            num_cores=NC, num_subcores=NS),
        compiler_params=pltpu.CompilerParams(
            needs_layout_passes=False, use_tc_tiling_on_sc=True),
        scratch_types=(
            [pltpu.VMEM((TBL_PAD // 128, 128), jnp.float32),
             pltpu.VMEM((N_IN, RH, CW), jnp.int32),
             pltpu.VMEM((N_IN, RH, CW), jnp.float32),
             pltpu.VMEM((N_IN, RH, CW), jnp.float32),
             pltpu.VMEM((N_OUT, RH, CW), jnp.float32)]
            + [pltpu.SemaphoreType.DMA] * (N_IN + N_OUT)),
    )(sig_tbl, idx_t, rl_t, kge_t)
    return out_t.T


# speedup vs baseline: 562.4991x; 1.0218x over previous
import functools

import jax
import jax.numpy as jnp
from jax import lax
from jax.experimental import pallas as pl
from jax.experimental.pallas import tpu as pltpu
from jax.experimental.pallas import tpu_sc as plsc

N_PRED = 100000
TBL_PAD = 100352
NC = 2
NS = 16
NW = NC * NS
LANES = 16
RH = 8
CW = 256
N_IN = 4
N_OUT = 2


def _sigmoid_body(x_ref, o_ref):
    o_ref[...] = jax.nn.sigmoid(x_ref[...])


def _bridge_body(tbl_hbm, idx_hbm, rl_hbm, kge_hbm, out_hbm,
                 tbl_v, idx_v, rl_v, kge_v, out_v,
                 in_sems, out_sems, n_per, ncg_shift):
    wid = lax.axis_index("s") * NC + lax.axis_index("c")
    base = wid * n_per
    ncg_mask = (1 << ncg_shift) - 1

    def slab(g):
        q = base + g
        return (q >> ncg_shift) * RH, (q & ncg_mask) * CW

    def start_in(g, slot):
        r0, c0 = slab(g)
        pltpu.async_copy(idx_hbm.at[pl.ds(r0, RH), pl.ds(c0, CW)],
                         idx_v.at[slot], in_sems[slot])
        pltpu.async_copy(rl_hbm.at[pl.ds(r0, RH), pl.ds(c0, CW)],
                         rl_v.at[slot], in_sems[slot])
        pltpu.async_copy(kge_hbm.at[pl.ds(r0, RH), pl.ds(c0, CW)],
                         kge_v.at[slot], in_sems[slot])

    def wait_in(slot):
        pltpu.make_async_copy(idx_hbm.at[pl.ds(0, RH), pl.ds(0, CW)],
                              idx_v.at[slot], in_sems[slot]).wait()
        pltpu.make_async_copy(rl_hbm.at[pl.ds(0, RH), pl.ds(0, CW)],
                              rl_v.at[slot], in_sems[slot]).wait()
        pltpu.make_async_copy(kge_hbm.at[pl.ds(0, RH), pl.ds(0, CW)],
                              kge_v.at[slot], in_sems[slot]).wait()

    def start_out(g, slot):
        r0, c0 = slab(g)
        pltpu.async_copy(out_v.at[slot],
                         out_hbm.at[pl.ds(r0, RH), pl.ds(c0, CW)],
                         out_sems[slot])

    def wait_out(slot):
        pltpu.make_async_copy(out_v.at[slot],
                              out_hbm.at[pl.ds(0, RH), pl.ds(0, CW)],
                              out_sems[slot]).wait()

    def compute(in_slot, out_slot):
        @plsc.parallel_loop(0, RH, 1)
        def _row(r):
            for c in range(0, CW, LANES):
                iv = idx_v[in_slot, r, pl.ds(c, LANES)]
                a = plsc.load_gather(tbl_v, [iv >> 7, iv & 127])
                rr = rl_v[in_slot, r, pl.ds(c, LANES)]
                kk = kge_v[in_slot, r, pl.ds(c, LANES)]
                out_v[out_slot, r, pl.ds(c, LANES)] = kk + a * (rr - kk)

    n_main = (n_per // N_IN) * N_IN
    n_tail = n_per - n_main
    assert n_tail < N_IN and n_tail % N_OUT == 0

    for s in range(N_IN - 1):
        start_in(s, s)
    pltpu.sync_copy(tbl_hbm, tbl_v)

    def group_body(p, carry):
        for b in range(N_IN):
            g = p * N_IN + b
            pre_g = g + N_IN - 1

            @pl.when(pre_g < n_per)
            def _():
                start_in(pre_g, (b + N_IN - 1) % N_IN)

            wait_in(b)

            @pl.when(g >= N_OUT)
            def _():
                wait_out(b % N_OUT)

            compute(b, b % N_OUT)
            start_out(g, b % N_OUT)
        return carry

    lax.fori_loop(0, n_main // N_IN, group_body, 0)

    for t in range(n_tail):
        g = n_main + t
        wait_in(g % N_IN)
        wait_out(g % N_OUT)
        compute(g % N_IN, g % N_OUT)
        start_out(g, g % N_OUT)

    for s in range(N_OUT):
        wait_out(s)


def kernel(rl_logprobs, kge_logprobs, pred_indices, alphas):
    B, K = rl_logprobs.shape
    assert K % RH == 0 and B % CW == 0
    ncg = B // CW
    ncg_shift = ncg.bit_length() - 1
    assert (1 << ncg_shift) == ncg
    n_chunks = (K // RH) * ncg
    n_per = n_chunks // NW
    assert n_per * NW == n_chunks

    alphas_p = jnp.pad(alphas, (0, TBL_PAD - N_PRED))
    sig_tbl = pl.pallas_call(
        _sigmoid_body,
        out_shape=jax.ShapeDtypeStruct((TBL_PAD // 128, 128), jnp.float32),
    )(alphas_p.reshape(TBL_PAD // 128, 128))

    idx_t = pred_indices.astype(jnp.int32).T
    rl_t = rl_logprobs.T
    kge_t = kge_logprobs.T

    def body(tbl_hbm, idx_hbm, rl_hbm, kge_hbm, out_hbm,
             tbl_v, idx_v, rl_v, kge_v, out_v, *sems):
        _bridge_body(tbl_hbm, idx_hbm, rl_hbm, kge_hbm, out_hbm,
                     tbl_v, idx_v, rl_v, kge_v, out_v,
                     sems[:N_IN], sems[N_IN:], n_per, ncg_shift)

    out_t = pl.kernel(
        body,
        out_type=jax.ShapeDtypeStruct((K, B), jnp.float32),
        mesh=plsc.VectorSubcoreMesh(
            core_axis_name="c", subcore_axis_name="s",
            num_cores=NC, num_subcores=NS),
        compiler_params=pltpu.CompilerParams(
            needs_layout_passes=False, use_tc_tiling_on_sc=True),
        scratch_types=(
            [pltpu.VMEM((TBL_PAD // 128, 128), jnp.float32),
             pltpu.VMEM((N_IN, RH, CW), jnp.int32),
             pltpu.VMEM((N_IN, RH, CW), jnp.float32),
             pltpu.VMEM((N_IN, RH, CW), jnp.float32),
             pltpu.VMEM((N_OUT, RH, CW), jnp.float32)]
            + [pltpu.SemaphoreType.DMA] * (N_IN + N_OUT)),
    )(sig_tbl, idx_t, rl_t, kge_t)
    return out_t.T


# speedup vs baseline: 601.2759x; 1.0689x over previous
import functools

import jax
import jax.numpy as jnp
from jax import lax
from jax.experimental import pallas as pl
from jax.experimental.pallas import tpu as pltpu
from jax.experimental.pallas import tpu_sc as plsc

N_PRED = 100000
TBL_PAD = 100352
NC = 2
NS = 16
NW = NC * NS
LANES = 16
RH = 8
CW = 256
N_IN = 4
N_OUT = 2


def _sigmoid_body(x_ref, o_ref):
    o_ref[...] = jax.nn.sigmoid(x_ref[...])


def _bridge_body(tbl_hbm, idx_hbm, rl_hbm, kge_hbm, out_hbm,
                 tbl_v, idx_v, rl_v, kge_v, out_v,
                 in_sem, out_sem, n_per, ncg_shift):
    wid = lax.axis_index("s") * NC + lax.axis_index("c")
    base = wid * n_per
    ncg_mask = (1 << ncg_shift) - 1

    def slab(g):
        q = base + g
        return (q >> ncg_shift) * RH, (q & ncg_mask) * CW

    def start_in(g, slot):
        r0, c0 = slab(g)
        pltpu.async_copy(idx_hbm.at[pl.ds(r0, RH), pl.ds(c0, CW)],
                         idx_v.at[slot], in_sem.at[slot])
        pltpu.async_copy(rl_hbm.at[pl.ds(r0, RH), pl.ds(c0, CW)],
                         rl_v.at[slot], in_sem.at[slot])
        pltpu.async_copy(kge_hbm.at[pl.ds(r0, RH), pl.ds(c0, CW)],
                         kge_v.at[slot], in_sem.at[slot])

    def wait_in(slot):
        pltpu.make_async_copy(idx_hbm.at[pl.ds(0, RH), pl.ds(0, CW)],
                              idx_v.at[slot], in_sem.at[slot]).wait()
        pltpu.make_async_copy(rl_hbm.at[pl.ds(0, RH), pl.ds(0, CW)],
                              rl_v.at[slot], in_sem.at[slot]).wait()
        pltpu.make_async_copy(kge_hbm.at[pl.ds(0, RH), pl.ds(0, CW)],
                              kge_v.at[slot], in_sem.at[slot]).wait()

    def start_out(g, slot):
        r0, c0 = slab(g)
        pltpu.async_copy(out_v.at[slot],
                         out_hbm.at[pl.ds(r0, RH), pl.ds(c0, CW)],
                         out_sem.at[slot])

    def wait_out(slot):
        pltpu.make_async_copy(out_v.at[slot],
                              out_hbm.at[pl.ds(0, RH), pl.ds(0, CW)],
                              out_sem.at[slot]).wait()

    def compute(in_slot, out_slot):
        @plsc.parallel_loop(0, RH * CW, LANES, unroll=2)
        def _vec(o):
            r = o >> 8
            c = o & (CW - 1)
            iv = idx_v[in_slot, r, pl.ds(c, LANES)]
            a = plsc.load_gather(tbl_v, [iv >> 7, iv & 127])
            rr = rl_v[in_slot, r, pl.ds(c, LANES)]
            kk = kge_v[in_slot, r, pl.ds(c, LANES)]
            out_v[out_slot, r, pl.ds(c, LANES)] = kk + a * (rr - kk)

    for s in range(N_IN - 1):
        start_in(s, s)
    pltpu.sync_copy(tbl_hbm, tbl_v)

    def chunk_body(g, carry):
        pre_g = g + N_IN - 1

        @pl.when(pre_g < n_per)
        def _():
            start_in(pre_g, pre_g & (N_IN - 1))

        wait_in(g & (N_IN - 1))

        @pl.when(g >= N_OUT)
        def _():
            wait_out(g & (N_OUT - 1))

        compute(g & (N_IN - 1), g & (N_OUT - 1))
        start_out(g, g & (N_OUT - 1))
        return carry

    lax.fori_loop(0, n_per, chunk_body, 0)
    for s in range(N_OUT):
        wait_out(s)


def kernel(rl_logprobs, kge_logprobs, pred_indices, alphas):
    B, K = rl_logprobs.shape
    assert K % RH == 0 and B % CW == 0
    ncg = B // CW
    ncg_shift = ncg.bit_length() - 1
    assert (1 << ncg_shift) == ncg
    n_chunks = (K // RH) * ncg
    n_per = n_chunks // NW
    assert n_per * NW == n_chunks

    alphas_p = jnp.pad(alphas, (0, TBL_PAD - N_PRED))
    sig_tbl = pl.pallas_call(
        _sigmoid_body,
        out_shape=jax.ShapeDtypeStruct((TBL_PAD // 128, 128), jnp.float32),
    )(alphas_p.reshape(TBL_PAD // 128, 128))

    idx_t = pred_indices.astype(jnp.int32).T
    rl_t = rl_logprobs.T
    kge_t = kge_logprobs.T

    body = functools.partial(_bridge_body, n_per=n_per, ncg_shift=ncg_shift)
    out_t = pl.kernel(
        body,
        out_type=jax.ShapeDtypeStruct((K, B), jnp.float32),
        mesh=plsc.VectorSubcoreMesh(
            core_axis_name="c", subcore_axis_name="s",
            num_cores=NC, num_subcores=NS),
        compiler_params=pltpu.CompilerParams(
            needs_layout_passes=False, use_tc_tiling_on_sc=True),
        scratch_types=[
            pltpu.VMEM((TBL_PAD // 128, 128), jnp.float32),
            pltpu.VMEM((N_IN, RH, CW), jnp.int32),
            pltpu.VMEM((N_IN, RH, CW), jnp.float32),
            pltpu.VMEM((N_IN, RH, CW), jnp.float32),
            pltpu.VMEM((N_OUT, RH, CW), jnp.float32),
            pltpu.SemaphoreType.DMA((N_IN,)),
            pltpu.SemaphoreType.DMA((N_OUT,)),
        ],
    )(sig_tbl, idx_t, rl_t, kge_t)
    return out_t.T
